# baseline (speedup 1.0000x reference)
import functools

import jax
import jax.numpy as jnp
from jax import lax
from jax.experimental import pallas as pl
from jax.experimental.pallas import tpu as pltpu
from jax.experimental.pallas import tpu_sc as plsc

D = 64
B = 16384
W = 2 * D

_INFO = plsc.get_sparse_core_info()
NC = _INFO.num_cores
NS = _INFO.num_subcores
L = _INFO.num_lanes
NW = NC * NS
BPW = B // NW
CHUNK = 128
PASS = 256
NPASS = BPW // PASS
NGROUP = PASS // L

_MAGIC = 0x5F3759DF
_INV_EPS = 1e8


def _rsqrt16(x):
    i = plsc.bitcast(x, jnp.int32)
    i = jnp.full((L,), _MAGIC, jnp.int32) - (i >> 1)
    y = plsc.bitcast(i, jnp.float32)
    half_x = x * 0.5
    for _ in range(3):
        y = y * (1.5 - half_x * y * y)
    return jnp.minimum(y, jnp.full((L,), _INV_EPS, jnp.float32))


def _sc_body(uidx_hbm, iidx_hbm, utab_hbm, itab_hbm, out_hbm,
             uidx_v, iidx_v, usr_v, isr_v, urows_v, irows_v, out_v, sem):
    wid = lax.axis_index("s") * NC + lax.axis_index("c")
    base = wid * BPW

    pltpu.sync_copy(uidx_hbm.at[pl.ds(base, BPW)], uidx_v)
    pltpu.sync_copy(iidx_hbm.at[pl.ds(base, BPW)], iidx_v)
    for g in range(BPW // L):
        usr_v[pl.ds(g * L, L)] = uidx_v[pl.ds(g * L, L)] >> 1
        isr_v[pl.ds(g * L, L)] = iidx_v[pl.ds(g * L, L)] >> 1

    lane = lax.iota(jnp.int32, L)
    one = jnp.full((L,), 1, jnp.int32)

    for p in range(NPASS):
        copies = []
        for c in range(PASS // CHUNK):
            off = p * PASS + c * CHUNK
            copies.append(pltpu.async_copy(
                utab_hbm.at[usr_v.at[pl.ds(off, CHUNK)]],
                urows_v.at[pl.ds(c * CHUNK, CHUNK)], sem))
            copies.append(pltpu.async_copy(
                itab_hbm.at[isr_v.at[pl.ds(off, CHUNK)]],
                irows_v.at[pl.ds(c * CHUNK, CHUNK)], sem))
        for cp in copies:
            cp.wait()

        for g in range(NGROUP):
            row_idx = lane + g * L
            ui = uidx_v[pl.ds(p * PASS + g * L, L)]
            ii = iidx_v[pl.ds(p * PASS + g * L, L)]
            ucol0 = (ui & one) * D
            icol0 = (ii & one) * D
            dot = jnp.zeros((L,), jnp.float32)
            uu = jnp.zeros((L,), jnp.float32)
            vv = jnp.zeros((L,), jnp.float32)
            for k in range(D):
                u = plsc.load_gather(urows_v, [row_idx, ucol0 + k])
                v = plsc.load_gather(irows_v, [row_idx, icol0 + k])
                dot = dot + u * v
                uu = uu + u * u
                vv = vv + v * v
            res = dot * _rsqrt16(uu) * _rsqrt16(vv)
            out_v[pl.ds(p * PASS + g * L, L)] = res

    pltpu.sync_copy(out_v, out_hbm.at[pl.ds(base, BPW)])


def kernel(user_indices, item_indices, user_table, item_table):
    n_users, d = user_table.shape
    n_items, _ = item_table.shape
    mesh = plsc.VectorSubcoreMesh(core_axis_name="c", subcore_axis_name="s")
    k = functools.partial(
        pl.kernel,
        mesh=mesh,
        out_type=jax.ShapeDtypeStruct((B,), jnp.float32),
        compiler_params=pltpu.CompilerParams(needs_layout_passes=False),
        scratch_types=[
            pltpu.VMEM((BPW,), jnp.int32),
            pltpu.VMEM((BPW,), jnp.int32),
            pltpu.VMEM((BPW,), jnp.int32),
            pltpu.VMEM((BPW,), jnp.int32),
            pltpu.VMEM((PASS, W), jnp.float32),
            pltpu.VMEM((PASS, W), jnp.float32),
            pltpu.VMEM((BPW,), jnp.float32),
            pltpu.SemaphoreType.DMA,
        ],
    )(_sc_body)
    return k(user_indices.astype(jnp.int32),
             item_indices.astype(jnp.int32),
             user_table.reshape(n_users * d // W, W),
             item_table.reshape(n_items * d // W, W))


# speedup vs baseline: 1.5442x; 1.5442x over previous
import functools

import jax
import jax.numpy as jnp
from jax import lax
from jax.experimental import pallas as pl
from jax.experimental.pallas import tpu as pltpu
from jax.experimental.pallas import tpu_sc as plsc

D = 64
B = 16384
SUB = 8

_INFO = plsc.get_sparse_core_info()
NC = _INFO.num_cores
NS = _INFO.num_subcores
L = _INFO.num_lanes
NW = NC * NS
BPW = B // NW
NPASS = BPW // L
NBUF = 2

_MAGIC = 0x5F3759DF
_INV_EPS = 1e8


def _rsqrt16(x):
    i = plsc.bitcast(x, jnp.int32)
    i = jnp.full((L,), _MAGIC, jnp.int32) - (i >> 1)
    y = plsc.bitcast(i, jnp.float32)
    half_x = x * 0.5
    for _ in range(3):
        y = y * (1.5 - half_x * y * y)
    return jnp.minimum(y, jnp.full((L,), _INV_EPS, jnp.float32))


def _sc_body(uidx_hbm, iidx_hbm, utab_hbm, itab_hbm, out_hbm,
             uidx_v, iidx_v, ubuf_v, ibuf_v, out_v, sem):
    wid = lax.axis_index("s") * NC + lax.axis_index("c")
    base = wid * BPW

    pltpu.sync_copy(uidx_hbm.at[pl.ds(base, BPW)], uidx_v)
    pltpu.sync_copy(iidx_hbm.at[pl.ds(base, BPW)], iidx_v)

    def fire(p, slot):
        uvec = uidx_v[pl.ds(p * L, L)]
        ivec = iidx_v[pl.ds(p * L, L)]
        for j in range(L):
            ur = uvec[j]
            ir = ivec[j]
            pltpu.async_copy(utab_hbm.at[pl.ds((ur >> 3) * SUB, SUB), :],
                             ubuf_v.at[slot, j], sem)
            pltpu.async_copy(itab_hbm.at[pl.ds((ir >> 3) * SUB, SUB), :],
                             ibuf_v.at[slot, j], sem)

    def drain(p, slot):
        uvec = uidx_v[pl.ds(p * L, L)]
        ivec = iidx_v[pl.ds(p * L, L)]
        for j in range(L):
            ur = uvec[j]
            ir = ivec[j]
            pltpu.make_async_copy(
                utab_hbm.at[pl.ds((ur >> 3) * SUB, SUB), :],
                ubuf_v.at[slot, j], sem).wait()
            pltpu.make_async_copy(
                itab_hbm.at[pl.ds((ir >> 3) * SUB, SUB), :],
                ibuf_v.at[slot, j], sem).wait()

    fire(0, 0)

    lane = lax.iota(jnp.int32, L)
    seven = jnp.full((L,), 7, jnp.int32)

    def step(p, carry):
        slot = lax.rem(p, NBUF)

        @pl.when(p + 1 < NPASS)
        def _prefetch():
            fire(p + 1, lax.rem(p + 1, NBUF))

        drain(p, slot)
        usub = uidx_v[pl.ds(p * L, L)] & seven
        isub = iidx_v[pl.ds(p * L, L)] & seven
        ubuf = ubuf_v.at[slot]
        ibuf = ibuf_v.at[slot]
        dot = jnp.zeros((L,), jnp.float32)
        uu = jnp.zeros((L,), jnp.float32)
        vv = jnp.zeros((L,), jnp.float32)
        for k in range(D):
            kv = jnp.full((L,), k, jnp.int32)
            u = plsc.load_gather(ubuf, [lane, usub, kv])
            v = plsc.load_gather(ibuf, [lane, isub, kv])
            dot = dot + u * v
            uu = uu + u * u
            vv = vv + v * v
        res = dot * _rsqrt16(uu) * _rsqrt16(vv)
        out_v[pl.ds(p * L, L)] = res
        return carry

    lax.fori_loop(0, NPASS, step, 0)

    pltpu.sync_copy(out_v, out_hbm.at[pl.ds(base, BPW)])


def kernel(user_indices, item_indices, user_table, item_table):
    mesh = plsc.VectorSubcoreMesh(core_axis_name="c", subcore_axis_name="s")
    k = functools.partial(
        pl.kernel,
        mesh=mesh,
        out_type=jax.ShapeDtypeStruct((B,), jnp.float32),
        compiler_params=pltpu.CompilerParams(needs_layout_passes=False),
        scratch_types=[
            pltpu.VMEM((BPW,), jnp.int32),
            pltpu.VMEM((BPW,), jnp.int32),
            pltpu.VMEM((NBUF, L, SUB, D), jnp.float32),
            pltpu.VMEM((NBUF, L, SUB, D), jnp.float32),
            pltpu.VMEM((BPW,), jnp.float32),
            pltpu.SemaphoreType.DMA,
        ],
    )(_sc_body)
    return k(user_indices.astype(jnp.int32),
             item_indices.astype(jnp.int32), user_table, item_table)


# speedup vs baseline: 3.1376x; 2.0319x over previous
import functools

import jax
import jax.numpy as jnp
from jax import lax
from jax.experimental import pallas as pl
from jax.experimental.pallas import tpu as pltpu
from jax.experimental.pallas import tpu_sc as plsc

D = 64
B = 16384
TW = 128

_INFO = plsc.get_sparse_core_info()
NC = _INFO.num_cores
NS = _INFO.num_subcores
L = _INFO.num_lanes
NW = NC * NS
BPW = B // NW
NGROUP = BPW // L
NBUF = 4
AHEAD = NBUF - 1

_MAGIC = 0x5F3759DF
_INV_EPS = 1e8


def _rsqrt16(x):
    i = plsc.bitcast(x, jnp.int32)
    i = jnp.full((L,), _MAGIC, jnp.int32) - (i >> 1)
    y = plsc.bitcast(i, jnp.float32)
    half_x = x * 0.5
    for _ in range(3):
        y = y * (1.5 - half_x * y * y)
    return jnp.minimum(y, jnp.full((L,), _INV_EPS, jnp.float32))


def _sc_body(uidx_hbm, iidx_hbm, utab_hbm, itab_hbm, out_hbm,
             uidx_v, iidx_v, ubuf_v, ibuf_v, out_v, sem):
    wid = lax.axis_index("s") * NC + lax.axis_index("c")
    base = wid * BPW

    pltpu.sync_copy(uidx_hbm.at[pl.ds(base, BPW)], uidx_v)
    pltpu.sync_copy(iidx_hbm.at[pl.ds(base, BPW)], iidx_v)

    def fire(ur, ir, slot):
        pltpu.async_copy(utab_hbm.at[:, pl.ds((ur >> 7) * TW, TW)],
                         ubuf_v.at[slot], sem)
        pltpu.async_copy(itab_hbm.at[:, pl.ds((ir >> 7) * TW, TW)],
                         ibuf_v.at[slot], sem)

    def drain(ur, ir, slot):
        pltpu.make_async_copy(utab_hbm.at[:, pl.ds((ur >> 7) * TW, TW)],
                              ubuf_v.at[slot], sem).wait()
        pltpu.make_async_copy(itab_hbm.at[:, pl.ds((ir >> 7) * TW, TW)],
                              ibuf_v.at[slot], sem).wait()

    def group_vecs(g):
        return uidx_v[pl.ds(g * L, L)], iidx_v[pl.ds(g * L, L)]

    uvec0, ivec0 = group_vecs(0)
    for j in range(AHEAD):
        fire(uvec0[j], ivec0[j], j)

    lane = lax.iota(jnp.int32, L)
    ones = jnp.full((L,), 1, jnp.int32)

    def step(g, carry):
        uvec, ivec = group_vecs(g)
        uvec_n, ivec_n = group_vecs(lax.rem(g + 1, NGROUP))
        dacc = jnp.zeros((L,), jnp.float32)
        uacc = jnp.zeros((L,), jnp.float32)
        vacc = jnp.zeros((L,), jnp.float32)
        for j in range(L):
            slot = (j + AHEAD) % NBUF

            if j + AHEAD < L:
                fire(uvec[j + AHEAD], ivec[j + AHEAD], slot)
            else:
                @pl.when(g + 1 < NGROUP)
                def _pref():
                    fire(uvec_n[j + AHEAD - L], ivec_n[j + AHEAD - L], slot)

            drain(uvec[j], ivec[j], j % NBUF)
            cu = ones * (uvec[j] & (TW - 1))
            ci = ones * (ivec[j] & (TW - 1))
            ubuf = ubuf_v.at[j % NBUF]
            ibuf = ibuf_v.at[j % NBUF]
            dot = jnp.zeros((L,), jnp.float32)
            uu = jnp.zeros((L,), jnp.float32)
            vv = jnp.zeros((L,), jnp.float32)
            for q in range(D // L):
                kv = lane + q * L
                u = plsc.load_gather(ubuf, [kv, cu])
                v = plsc.load_gather(ibuf, [kv, ci])
                dot = dot + u * v
                uu = uu + u * u
                vv = vv + v * v
            sel = lane == j
            dacc = jnp.where(sel, plsc.cumsum(dot)[L - 1], dacc)
            uacc = jnp.where(sel, plsc.cumsum(uu)[L - 1], uacc)
            vacc = jnp.where(sel, plsc.cumsum(vv)[L - 1], vacc)
        res = dacc * _rsqrt16(uacc) * _rsqrt16(vacc)
        out_v[pl.ds(g * L, L)] = res
        return carry

    lax.fori_loop(0, NGROUP, step, 0)

    pltpu.sync_copy(out_v, out_hbm.at[pl.ds(base, BPW)])


def kernel(user_indices, item_indices, user_table, item_table):
    mesh = plsc.VectorSubcoreMesh(core_axis_name="c", subcore_axis_name="s")
    k = functools.partial(
        pl.kernel,
        mesh=mesh,
        out_type=jax.ShapeDtypeStruct((B,), jnp.float32),
        compiler_params=pltpu.CompilerParams(needs_layout_passes=False),
        scratch_types=[
            pltpu.VMEM((BPW,), jnp.int32),
            pltpu.VMEM((BPW,), jnp.int32),
            pltpu.VMEM((NBUF, D, TW), jnp.float32),
            pltpu.VMEM((NBUF, D, TW), jnp.float32),
            pltpu.VMEM((BPW,), jnp.float32),
            pltpu.SemaphoreType.DMA,
        ],
    )(_sc_body)
    return k(user_indices.astype(jnp.int32),
             item_indices.astype(jnp.int32), user_table.T, item_table.T)
